# baseline (speedup 1.0000x reference)
import functools

import numpy as np
import jax
import jax.numpy as jnp
from jax import lax
from jax.experimental import pallas as pl
from jax.experimental.pallas import tpu as pltpu
from jax.experimental.pallas import tpu_sc as plsc

NLEV = 8
SCALES = [16, 21, 28, 39, 52, 70, 95, 128]
_offs = [0]
for _s in SCALES:
    _offs.append(_offs[-1] + (_s + 1) ** 3)
OFFS = _offs
TOTAL = OFFS[-1]
FREQ = [float(2.0 ** l) for l in range(NLEV)]

NPTS = 100000
NW = 32
MBS = 16
PPW = 3200
NMB = PPW // MBS
NPAD = NW * PPW
OUTW = 3 + 6 * NLEV

BOUND_LO = np.float32(-1.0)
BOUND_HI = np.float32(np.float32(1.0) - np.float32(1e-6))

S0, S1, S2 = 0.99977134, -0.16582688, 0.00757417
C0, C1, C2, C3 = 0.99999528, -0.4999309, 0.04151171, -0.0012787
MAGIC = np.float32(1.5 * 2 ** 23)
INV_PI = float(1.0 / np.pi)
PI_HI = float(np.float32(np.pi))
PI_LO = float(np.float32(np.pi - np.float64(np.float32(np.pi))))


def _sincos(a):
    kf = a * INV_PI + MAGIC
    kbits = lax.bitcast_convert_type(kf, jnp.int32)
    kf = kf - MAGIC
    r = a - kf * PI_HI
    r = r - kf * PI_LO
    r2 = r * r
    ps = r * (S0 + r2 * (S1 + r2 * S2))
    pc = C0 + r2 * (C1 + r2 * (C2 + r2 * C3))
    sgn = lax.shift_left(lax.bitwise_and(kbits, 1), 31)
    s = lax.bitcast_convert_type(
        lax.bitwise_xor(lax.bitcast_convert_type(ps, jnp.int32), sgn), jnp.float32)
    c = lax.bitcast_convert_type(
        lax.bitwise_xor(lax.bitcast_convert_type(pc, jnp.int32), sgn), jnp.float32)
    return s, c


def _splat_i32(v):
    return jnp.full((16,), v, dtype=jnp.int32)


def _make_sc_call():
    mesh = plsc.VectorSubcoreMesh(core_axis_name="c", subcore_axis_name="s")

    @functools.partial(
        pl.kernel,
        mesh=mesh,
        out_type=jax.ShapeDtypeStruct((NPAD, OUTW), jnp.float32),
        compiler_params=pltpu.CompilerParams(
            needs_layout_passes=False, use_tc_tiling_on_sc=False),
        scratch_types=[
            pltpu.VMEM((3, PPW), jnp.float32),
            pltpu.VMEM((NLEV, 128), jnp.int32),
            pltpu.VMEM((NLEV, 128), jnp.int32),
            pltpu.VMEM((NLEV, 128, 8), jnp.float32),
            pltpu.VMEM((MBS, OUTW), jnp.float32),
            pltpu.SemaphoreType.DMA,
        ],
    )
    def sc_call(xyz_hbm, data_hbm, out_hbm, xbuf, idxbuf, pbuf, gbuf, obuf, sem):
        wid = lax.axis_index("s") * 2 + lax.axis_index("c")
        wbase = wid * PPW
        for i in range(3):
            pltpu.sync_copy(xyz_hbm.at[i, pl.ds(wbase, PPW)], xbuf.at[i])

        iota = lax.broadcasted_iota(jnp.int32, (16,), 0)

        def mb_body(m, _):
            base = m * MBS
            x = xbuf[0, pl.ds(base, MBS)]
            y = xbuf[1, pl.ds(base, MBS)]
            z = xbuf[2, pl.ds(base, MBS)]
            xn = (jnp.minimum(jnp.maximum(x, BOUND_LO), BOUND_HI) + 1.0) * 0.5
            yn = (jnp.minimum(jnp.maximum(y, BOUND_LO), BOUND_HI) + 1.0) * 0.5
            zn = (jnp.minimum(jnp.maximum(z, BOUND_LO), BOUND_HI) + 1.0) * 0.5

            copies = []
            lvl_cache = []
            for l in range(NLEV):
                scale = float(SCALES[l])
                sp1 = SCALES[l] + 1
                fx = xn * scale
                fy = yn * scale
                fz = zn * scale
                ix0 = lax.convert_element_type(fx, jnp.int32)
                iy0 = lax.convert_element_type(fy, jnp.int32)
                iz0 = lax.convert_element_type(fz, jnp.int32)
                ix1 = lax.convert_element_type(fx + 1.0, jnp.int32)
                iy1 = lax.convert_element_type(fy + 1.0, jnp.int32)
                iz1 = lax.convert_element_type(fz + 1.0, jnp.int32)
                ax0 = ix0 * (sp1 * sp1) + OFFS[l]
                ax1 = ix1 * (sp1 * sp1) + OFFS[l]
                by0 = iy0 * sp1
                by1 = iy1 * sp1
                for c in range(8):
                    ind = ((ax1 if (c & 4) else ax0)
                           + (by1 if (c & 2) else by0)
                           + (iz1 if (c & 1) else iz0))
                    idxbuf[l, pl.ds(c * 16, 16)] = lax.shift_right_arithmetic(ind, 1)
                    pbuf[l, pl.ds(c * 16, 16)] = lax.shift_left(
                        lax.bitwise_and(ind, 1), 2)
                cp = pltpu.make_async_copy(
                    data_hbm.at[idxbuf.at[l]], gbuf.at[l], sem)
                cp.start()
                copies.append(cp)
                tx = fx - lax.convert_element_type(ix0, jnp.float32)
                ty = fy - lax.convert_element_type(iy0, jnp.float32)
                tz = fz - lax.convert_element_type(iz0, jnp.float32)
                lvl_cache.append((tx, ty, tz))

            for l in range(NLEV):
                copies[l].wait()
                tx, ty, tz = lvl_cache[l]
                wx = (jnp.clip(1.0 - tx, 0.0, 1.0), jnp.clip(tx, 0.0, 1.0))
                wy = (jnp.clip(1.0 - ty, 0.0, 1.0), jnp.clip(ty, 0.0, 1.0))
                wz = (jnp.clip(1.0 - tz, 0.0, 1.0), jnp.clip(tz, 0.0, 1.0))
                asx = jnp.zeros((16,), jnp.float32)
                asy = jnp.zeros((16,), jnp.float32)
                asz = jnp.zeros((16,), jnp.float32)
                acx = jnp.zeros((16,), jnp.float32)
                acy = jnp.zeros((16,), jnp.float32)
                acz = jnp.zeros((16,), jnp.float32)
                lsplat = _splat_i32(l)
                for c in range(8):
                    w = (wx[(c >> 2) & 1] * wy[(c >> 1) & 1]) * wz[c & 1]
                    rows = iota + (c * 16)
                    cb = pbuf[l, pl.ds(c * 16, 16)]
                    vx = plsc.load_gather(gbuf, [lsplat, rows, cb])
                    vy = plsc.load_gather(gbuf, [lsplat, rows, cb + 1])
                    vz = plsc.load_gather(gbuf, [lsplat, rows, cb + 2])
                    sx, cx = _sincos(vx * FREQ[l])
                    sy, cy = _sincos(vy * FREQ[l])
                    sz, cz = _sincos(vz * FREQ[l])
                    asx = asx + w * sx
                    asy = asy + w * sy
                    asz = asz + w * sz
                    acx = acx + w * cx
                    acy = acy + w * cy
                    acz = acz + w * cz
                col = 3 + 6 * l
                plsc.store_scatter(obuf, [iota, _splat_i32(col + 0)], asx)
                plsc.store_scatter(obuf, [iota, _splat_i32(col + 1)], asy)
                plsc.store_scatter(obuf, [iota, _splat_i32(col + 2)], asz)
                plsc.store_scatter(obuf, [iota, _splat_i32(col + 3)], acx)
                plsc.store_scatter(obuf, [iota, _splat_i32(col + 4)], acy)
                plsc.store_scatter(obuf, [iota, _splat_i32(col + 5)], acz)

            plsc.store_scatter(obuf, [iota, _splat_i32(0)], x)
            plsc.store_scatter(obuf, [iota, _splat_i32(1)], y)
            plsc.store_scatter(obuf, [iota, _splat_i32(2)], z)
            pltpu.sync_copy(obuf, out_hbm.at[pl.ds(wbase + base, MBS)])
            return _

        lax.fori_loop(0, NMB, mb_body, None)

    return sc_call


_sc_call_cache = []


def kernel(xyz, data):
    if not _sc_call_cache:
        _sc_call_cache.append(_make_sc_call())
    xyz_t = jnp.pad(xyz, ((0, NPAD - NPTS), (0, 0))).T
    data_p = jnp.pad(data, ((0, 1), (0, 1))).reshape((TOTAL + 1) // 2, 8)
    out = _sc_call_cache[0](xyz_t, data_p)
    return out[:NPTS]


# speedup vs baseline: 1.0003x; 1.0003x over previous
import functools

import numpy as np
import jax
import jax.numpy as jnp
from jax import lax
from jax.experimental import pallas as pl
from jax.experimental.pallas import tpu as pltpu
from jax.experimental.pallas import tpu_sc as plsc

NLEV = 8
SCALES = [16, 21, 28, 39, 52, 70, 95, 128]
_offs = [0]
for _s in SCALES:
    _offs.append(_offs[-1] + (_s + 1) ** 3)
OFFS = _offs
TOTAL = OFFS[-1]
FREQ = [float(2.0 ** l) for l in range(NLEV)]

NPTS = 100000
NW = 32
MBS = 16
PPW = 3200
NMB = PPW // MBS
NPAD = NW * PPW
OUTW = 3 + 6 * NLEV

BOUND_LO = np.float32(-1.0)
BOUND_HI = np.float32(np.float32(1.0) - np.float32(1e-6))

S0, S1, S2 = 0.99977134, -0.16582688, 0.00757417
C0, C1, C2, C3 = 0.99999528, -0.4999309, 0.04151171, -0.0012787
MAGIC = np.float32(1.5 * 2 ** 23)
INV_PI = float(1.0 / np.pi)
PI_HI = float(np.float32(np.pi))
PI_LO = float(np.float32(np.pi - np.float64(np.float32(np.pi))))


def _sincos(a):
    kf = a * INV_PI + MAGIC
    kbits = lax.bitcast_convert_type(kf, jnp.int32)
    kf = kf - MAGIC
    r = a - kf * PI_HI
    r = r - kf * PI_LO
    r2 = r * r
    ps = r * (S0 + r2 * (S1 + r2 * S2))
    pc = C0 + r2 * (C1 + r2 * (C2 + r2 * C3))
    sgn = lax.shift_left(lax.bitwise_and(kbits, 1), 31)
    s = lax.bitcast_convert_type(
        lax.bitwise_xor(lax.bitcast_convert_type(ps, jnp.int32), sgn), jnp.float32)
    c = lax.bitcast_convert_type(
        lax.bitwise_xor(lax.bitcast_convert_type(pc, jnp.int32), sgn), jnp.float32)
    return s, c


def _splat_i32(v):
    return jnp.full((16,), v, dtype=jnp.int32)


def _make_sc_call():
    mesh = plsc.VectorSubcoreMesh(core_axis_name="c", subcore_axis_name="s")

    @functools.partial(
        pl.kernel,
        mesh=mesh,
        out_type=jax.ShapeDtypeStruct((NPAD, OUTW), jnp.float32),
        compiler_params=pltpu.CompilerParams(
            needs_layout_passes=False, use_tc_tiling_on_sc=False),
        scratch_types=[
            pltpu.VMEM((3, PPW), jnp.float32),
            pltpu.VMEM((NLEV, 128), jnp.int32),
            pltpu.VMEM((NLEV, 128), jnp.int32),
            pltpu.VMEM((NLEV, 128, 8), jnp.float32),
            pltpu.VMEM((MBS, OUTW), jnp.float32),
            pltpu.SemaphoreType.DMA,
        ],
    )
    def sc_call(xyz_hbm, data_hbm, out_hbm, xbuf, idxbuf, pbuf, gbuf, obuf, sem):
        wid = lax.axis_index("s") * 2 + lax.axis_index("c")
        wbase = wid * PPW
        for i in range(3):
            pltpu.sync_copy(xyz_hbm.at[i, pl.ds(wbase, PPW)], xbuf.at[i])

        iota = lax.broadcasted_iota(jnp.int32, (16,), 0)

        def mb_body(m, _):
            base = m * MBS
            x = xbuf[0, pl.ds(base, MBS)]
            y = xbuf[1, pl.ds(base, MBS)]
            z = xbuf[2, pl.ds(base, MBS)]
            xn = (jnp.minimum(jnp.maximum(x, BOUND_LO), BOUND_HI) + 1.0) * 0.5
            yn = (jnp.minimum(jnp.maximum(y, BOUND_LO), BOUND_HI) + 1.0) * 0.5
            zn = (jnp.minimum(jnp.maximum(z, BOUND_LO), BOUND_HI) + 1.0) * 0.5

            copies = []
            lvl_cache = []
            for l in range(NLEV):
                scale = float(SCALES[l])
                sp1 = SCALES[l] + 1
                fx = xn * scale
                fy = yn * scale
                fz = zn * scale
                ix0 = lax.convert_element_type(fx, jnp.int32)
                iy0 = lax.convert_element_type(fy, jnp.int32)
                iz0 = lax.convert_element_type(fz, jnp.int32)
                ix1 = lax.convert_element_type(fx + 1.0, jnp.int32)
                iy1 = lax.convert_element_type(fy + 1.0, jnp.int32)
                iz1 = lax.convert_element_type(fz + 1.0, jnp.int32)
                ax0 = ix0 * (sp1 * sp1) + OFFS[l]
                ax1 = ix1 * (sp1 * sp1) + OFFS[l]
                by0 = iy0 * sp1
                by1 = iy1 * sp1
                for c in range(8):
                    ind = ((ax1 if (c & 4) else ax0)
                           + (by1 if (c & 2) else by0)
                           + (iz1 if (c & 1) else iz0))
                    idxbuf[l, pl.ds(c * 16, 16)] = lax.shift_right_arithmetic(ind, 1)
                    pbuf[l, pl.ds(c * 16, 16)] = lax.shift_left(
                        lax.bitwise_and(ind, 1), 2)
                cp = pltpu.make_async_copy(
                    data_hbm.at[idxbuf.at[l]], gbuf.at[l], sem)
                cp.start()
                copies.append(cp)
                tx = fx - lax.convert_element_type(ix0, jnp.float32)
                ty = fy - lax.convert_element_type(iy0, jnp.float32)
                tz = fz - lax.convert_element_type(iz0, jnp.float32)
                lvl_cache.append((tx, ty, tz))

            for l in range(NLEV):
                copies[l].wait()
                tx, ty, tz = lvl_cache[l]
                wx = (jnp.clip(1.0 - tx, 0.0, 1.0), jnp.clip(tx, 0.0, 1.0))
                wy = (jnp.clip(1.0 - ty, 0.0, 1.0), jnp.clip(ty, 0.0, 1.0))
                wz = (jnp.clip(1.0 - tz, 0.0, 1.0), jnp.clip(tz, 0.0, 1.0))
                asx = jnp.zeros((16,), jnp.float32)
                asy = jnp.zeros((16,), jnp.float32)
                asz = jnp.zeros((16,), jnp.float32)
                acx = jnp.zeros((16,), jnp.float32)
                acy = jnp.zeros((16,), jnp.float32)
                acz = jnp.zeros((16,), jnp.float32)
                lsplat = _splat_i32(l)
                for c in range(8):
                    w = (wx[(c >> 2) & 1] * wy[(c >> 1) & 1]) * wz[c & 1]
                    rows = iota + (c * 16)
                    cb = pbuf[l, pl.ds(c * 16, 16)]
                    vx = plsc.load_gather(gbuf, [lsplat, rows, cb])
                    vy = plsc.load_gather(gbuf, [lsplat, rows, cb + 1])
                    vz = plsc.load_gather(gbuf, [lsplat, rows, cb + 2])
                    sx, cx = _sincos(vx * FREQ[l])
                    sy, cy = _sincos(vy * FREQ[l])
                    sz, cz = _sincos(vz * FREQ[l])
                    asx = asx + w * sx
                    asy = asy + w * sy
                    asz = asz + w * sz
                    acx = acx + w * cx
                    acy = acy + w * cy
                    acz = acz + w * cz
                col = 3 + 6 * l
                plsc.store_scatter(obuf, [iota, _splat_i32(col + 0)], asx)
                plsc.store_scatter(obuf, [iota, _splat_i32(col + 1)], asy)
                plsc.store_scatter(obuf, [iota, _splat_i32(col + 2)], asz)
                plsc.store_scatter(obuf, [iota, _splat_i32(col + 3)], acx)
                plsc.store_scatter(obuf, [iota, _splat_i32(col + 4)], acy)
                plsc.store_scatter(obuf, [iota, _splat_i32(col + 5)], acz)

            plsc.store_scatter(obuf, [iota, _splat_i32(0)], x)
            plsc.store_scatter(obuf, [iota, _splat_i32(1)], y)
            plsc.store_scatter(obuf, [iota, _splat_i32(2)], z)
            pltpu.sync_copy(obuf, out_hbm.at[pl.ds(wbase + base, MBS)])
            return _

        lax.fori_loop(0, NMB, mb_body, None)

    return sc_call


_sc_call_cache = []


def kernel(xyz, data):
    if not _sc_call_cache:
        _sc_call_cache.append(_make_sc_call())
    xyz_t = jnp.pad(xyz, ((0, NPAD - NPTS), (0, 0))).T
    flat = jnp.pad(data, ((0, 1), (0, 1))).reshape(-1)
    flat = lax.optimization_barrier(flat)
    data_p = flat.reshape((TOTAL + 1) // 2, 8)
    out = _sc_call_cache[0](xyz_t, data_p)
    return out[:NPTS]


# speedup vs baseline: 2.3112x; 2.3104x over previous
import functools

import numpy as np
import jax
import jax.numpy as jnp
from jax import lax
from jax.experimental import pallas as pl
from jax.experimental.pallas import tpu as pltpu
from jax.experimental.pallas import tpu_sc as plsc

NLEV = 8
SCALES = [16, 21, 28, 39, 52, 70, 95, 128]
_offs = [0]
for _s in SCALES:
    _offs.append(_offs[-1] + (_s + 1) ** 3)
OFFS = _offs
TOTAL = OFFS[-1]
FREQ = [float(2.0 ** l) for l in range(NLEV)]

NPTS = 100000
NW = 32
MBS = 16
PPW = 3200
NMB = PPW // MBS
NPAD = NW * PPW
OUTW = 3 + 6 * NLEV

BOUND_LO = np.float32(-1.0)
BOUND_HI = np.float32(np.float32(1.0) - np.float32(1e-6))

S0, S1, S2 = 0.99977134, -0.16582688, 0.00757417
C0, C1, C2, C3 = 0.99999528, -0.4999309, 0.04151171, -0.0012787
MAGIC = np.float32(1.5 * 2 ** 23)
INV_PI = float(1.0 / np.pi)
PI_HI = float(np.float32(np.pi))
PI_LO = float(np.float32(np.pi - np.float64(np.float32(np.pi))))


def _sincos(a):
    kf = a * INV_PI + MAGIC
    kbits = lax.bitcast_convert_type(kf, jnp.int32)
    kf = kf - MAGIC
    r = a - kf * PI_HI
    r = r - kf * PI_LO
    r2 = r * r
    ps = r * (S0 + r2 * (S1 + r2 * S2))
    pc = C0 + r2 * (C1 + r2 * (C2 + r2 * C3))
    sgn = lax.shift_left(lax.bitwise_and(kbits, 1), 31)
    s = lax.bitcast_convert_type(
        lax.bitwise_xor(lax.bitcast_convert_type(ps, jnp.int32), sgn), jnp.float32)
    c = lax.bitcast_convert_type(
        lax.bitwise_xor(lax.bitcast_convert_type(pc, jnp.int32), sgn), jnp.float32)
    return s, c


def _splat_i32(v):
    return jnp.full((16,), v, dtype=jnp.int32)


def _make_sc_call():
    mesh = plsc.VectorSubcoreMesh(core_axis_name="c", subcore_axis_name="s")

    @functools.partial(
        pl.kernel,
        mesh=mesh,
        out_type=jax.ShapeDtypeStruct((NPAD, OUTW), jnp.float32),
        compiler_params=pltpu.CompilerParams(
            needs_layout_passes=False, use_tc_tiling_on_sc=False),
        scratch_types=[
            pltpu.VMEM((3, PPW), jnp.float32),
            pltpu.VMEM((NLEV, 128), jnp.int32),
            pltpu.VMEM((NLEV, 128), jnp.int32),
            pltpu.VMEM((NLEV, 128, 8), jnp.float32),
            pltpu.VMEM((MBS, OUTW), jnp.float32),
            pltpu.SemaphoreType.DMA,
        ],
    )
    def sc_call(xyz_hbm, data_hbm, out_hbm, xbuf, idxbuf, pbuf, gbuf, obuf, sem):
        wid = lax.axis_index("s") * 2 + lax.axis_index("c")
        wbase = wid * PPW
        for i in range(3):
            pltpu.sync_copy(xyz_hbm.at[i, pl.ds(wbase, PPW)], xbuf.at[i])

        iota = lax.broadcasted_iota(jnp.int32, (16,), 0)

        def mb_body(m, _):
            base = m * MBS
            x = xbuf[0, pl.ds(base, MBS)]
            y = xbuf[1, pl.ds(base, MBS)]
            z = xbuf[2, pl.ds(base, MBS)]
            xn = (jnp.minimum(jnp.maximum(x, BOUND_LO), BOUND_HI) + 1.0) * 0.5
            yn = (jnp.minimum(jnp.maximum(y, BOUND_LO), BOUND_HI) + 1.0) * 0.5
            zn = (jnp.minimum(jnp.maximum(z, BOUND_LO), BOUND_HI) + 1.0) * 0.5

            copies = []
            lvl_cache = []
            for l in range(NLEV):
                scale = float(SCALES[l])
                sp1 = SCALES[l] + 1
                fx = xn * scale
                fy = yn * scale
                fz = zn * scale
                ix0 = lax.convert_element_type(fx, jnp.int32)
                iy0 = lax.convert_element_type(fy, jnp.int32)
                iz0 = lax.convert_element_type(fz, jnp.int32)
                ix1 = lax.convert_element_type(fx + 1.0, jnp.int32)
                iy1 = lax.convert_element_type(fy + 1.0, jnp.int32)
                iz1 = lax.convert_element_type(fz + 1.0, jnp.int32)
                ax0 = ix0 * (sp1 * sp1) + OFFS[l]
                ax1 = ix1 * (sp1 * sp1) + OFFS[l]
                by0 = iy0 * sp1
                by1 = iy1 * sp1
                for c in range(8):
                    ind = ((ax1 if (c & 4) else ax0)
                           + (by1 if (c & 2) else by0)
                           + (iz1 if (c & 1) else iz0))
                    idxbuf[l, pl.ds(c * 16, 16)] = lax.shift_right_arithmetic(ind, 1)
                    pbuf[l, pl.ds(c * 16, 16)] = lax.shift_left(
                        lax.bitwise_and(ind, 1), 2)
                cp = pltpu.make_async_copy(
                    data_hbm.at[idxbuf.at[l]], gbuf.at[l], sem)
                cp.start()
                copies.append(cp)
                tx = fx - lax.convert_element_type(ix0, jnp.float32)
                ty = fy - lax.convert_element_type(iy0, jnp.float32)
                tz = fz - lax.convert_element_type(iz0, jnp.float32)
                lvl_cache.append((tx, ty, tz))

            for l in range(NLEV):
                copies[l].wait()
                tx, ty, tz = lvl_cache[l]
                wx = (jnp.clip(1.0 - tx, 0.0, 1.0), jnp.clip(tx, 0.0, 1.0))
                wy = (jnp.clip(1.0 - ty, 0.0, 1.0), jnp.clip(ty, 0.0, 1.0))
                wz = (jnp.clip(1.0 - tz, 0.0, 1.0), jnp.clip(tz, 0.0, 1.0))
                asx = jnp.zeros((16,), jnp.float32)
                asy = jnp.zeros((16,), jnp.float32)
                asz = jnp.zeros((16,), jnp.float32)
                acx = jnp.zeros((16,), jnp.float32)
                acy = jnp.zeros((16,), jnp.float32)
                acz = jnp.zeros((16,), jnp.float32)
                lsplat = _splat_i32(l)
                for c in range(8):
                    w = (wx[(c >> 2) & 1] * wy[(c >> 1) & 1]) * wz[c & 1]
                    rows = iota + (c * 16)
                    cb = pbuf[l, pl.ds(c * 16, 16)]
                    vx = plsc.load_gather(gbuf, [lsplat, rows, cb])
                    vy = plsc.load_gather(gbuf, [lsplat, rows, cb + 1])
                    vz = plsc.load_gather(gbuf, [lsplat, rows, cb + 2])
                    sx, cx = _sincos(vx * FREQ[l])
                    sy, cy = _sincos(vy * FREQ[l])
                    sz, cz = _sincos(vz * FREQ[l])
                    asx = asx + w * sx
                    asy = asy + w * sy
                    asz = asz + w * sz
                    acx = acx + w * cx
                    acy = acy + w * cy
                    acz = acz + w * cz
                col = 3 + 6 * l
                plsc.store_scatter(obuf, [iota, _splat_i32(col + 0)], asx)
                plsc.store_scatter(obuf, [iota, _splat_i32(col + 1)], asy)
                plsc.store_scatter(obuf, [iota, _splat_i32(col + 2)], asz)
                plsc.store_scatter(obuf, [iota, _splat_i32(col + 3)], acx)
                plsc.store_scatter(obuf, [iota, _splat_i32(col + 4)], acy)
                plsc.store_scatter(obuf, [iota, _splat_i32(col + 5)], acz)

            plsc.store_scatter(obuf, [iota, _splat_i32(0)], x)
            plsc.store_scatter(obuf, [iota, _splat_i32(1)], y)
            plsc.store_scatter(obuf, [iota, _splat_i32(2)], z)
            pltpu.sync_copy(obuf, out_hbm.at[pl.ds(wbase + base, MBS)])
            return _

        lax.fori_loop(0, NMB, mb_body, None)

    return sc_call


_sc_call_cache = []


def kernel(xyz, data):
    if not _sc_call_cache:
        _sc_call_cache.append(_make_sc_call())
    xyz_t = jnp.pad(xyz, ((0, NPAD - NPTS), (0, 0))).T
    one = xyz[0, 0] * 0.0 + 1.0
    flat = jnp.pad(data, ((0, 1), (0, 1))).reshape(-1) * one
    data_p = flat.reshape((TOTAL + 1) // 2, 8)
    out = _sc_call_cache[0](xyz_t, data_p)
    return out[:NPTS]


# speedup vs baseline: 8.7248x; 3.7750x over previous
import functools

import numpy as np
import jax
import jax.numpy as jnp
from jax import lax
from jax.experimental import pallas as pl
from jax.experimental.pallas import tpu as pltpu
from jax.experimental.pallas import tpu_sc as plsc

NLEV = 8
SCALES = [16, 21, 28, 39, 52, 70, 95, 128]
_offs = [0]
for _s in SCALES:
    _offs.append(_offs[-1] + (_s + 1) ** 3)
OFFS = _offs
TOTAL = OFFS[-1]
FREQ = [float(2.0 ** l) for l in range(NLEV)]

NPTS = 100000
NW = 32
MBS = 16
PPW = 3200
NMB = PPW // MBS
NPAD = NW * PPW
OUTW = 3 + 6 * NLEV

BOUND_LO = np.float32(-1.0)
BOUND_HI = np.float32(np.float32(1.0) - np.float32(1e-6))

S0, S1, S2 = 0.99977134, -0.16582688, 0.00757417
C0, C1, C2, C3 = 0.99999528, -0.4999309, 0.04151171, -0.0012787
MAGIC = np.float32(1.5 * 2 ** 23)
INV_PI = float(1.0 / np.pi)
PI_HI = float(np.float32(np.pi))
PI_LO = float(np.float32(np.pi - np.float64(np.float32(np.pi))))


def _sincos(a):
    kf = a * INV_PI + MAGIC
    kbits = lax.bitcast_convert_type(kf, jnp.int32)
    kf = kf - MAGIC
    r = a - kf * PI_HI
    r = r - kf * PI_LO
    r2 = r * r
    ps = r * (S0 + r2 * (S1 + r2 * S2))
    pc = C0 + r2 * (C1 + r2 * (C2 + r2 * C3))
    sgn = lax.shift_left(lax.bitwise_and(kbits, 1), 31)
    s = lax.bitcast_convert_type(
        lax.bitwise_xor(lax.bitcast_convert_type(ps, jnp.int32), sgn), jnp.float32)
    c = lax.bitcast_convert_type(
        lax.bitwise_xor(lax.bitcast_convert_type(pc, jnp.int32), sgn), jnp.float32)
    return s, c


def _splat_i32(v):
    return jnp.full((16,), v, dtype=jnp.int32)


def _make_sc_call():
    mesh = plsc.VectorSubcoreMesh(core_axis_name="c", subcore_axis_name="s")

    @functools.partial(
        pl.kernel,
        mesh=mesh,
        out_type=jax.ShapeDtypeStruct((NPAD, OUTW), jnp.float32),
        compiler_params=pltpu.CompilerParams(
            needs_layout_passes=False, use_tc_tiling_on_sc=False),
        scratch_types=[
            pltpu.VMEM((3, PPW), jnp.float32),
            pltpu.VMEM((NLEV, 128), jnp.int32),
            pltpu.VMEM((NLEV, 3, 128), jnp.float32),
            pltpu.VMEM((MBS, OUTW), jnp.float32),
            pltpu.SemaphoreType.DMA,
        ],
    )
    def sc_call(xyz_hbm, dx_hbm, dy_hbm, dz_hbm, out_hbm,
                xbuf, idxbuf, gbuf, obuf, sem):
        wid = lax.axis_index("s") * 2 + lax.axis_index("c")
        wbase = wid * PPW
        for i in range(3):
            pltpu.sync_copy(xyz_hbm.at[i, pl.ds(wbase, PPW)], xbuf.at[i])

        iota = lax.broadcasted_iota(jnp.int32, (16,), 0)

        def mb_body(m, _):
            base = m * MBS
            x = xbuf[0, pl.ds(base, MBS)]
            y = xbuf[1, pl.ds(base, MBS)]
            z = xbuf[2, pl.ds(base, MBS)]
            xn = (jnp.minimum(jnp.maximum(x, BOUND_LO), BOUND_HI) + 1.0) * 0.5
            yn = (jnp.minimum(jnp.maximum(y, BOUND_LO), BOUND_HI) + 1.0) * 0.5
            zn = (jnp.minimum(jnp.maximum(z, BOUND_LO), BOUND_HI) + 1.0) * 0.5

            copies = []
            lvl_cache = []
            for l in range(NLEV):
                scale = float(SCALES[l])
                sp1 = SCALES[l] + 1
                fx = xn * scale
                fy = yn * scale
                fz = zn * scale
                ix0 = lax.convert_element_type(fx, jnp.int32)
                iy0 = lax.convert_element_type(fy, jnp.int32)
                iz0 = lax.convert_element_type(fz, jnp.int32)
                ix1 = lax.convert_element_type(fx + 1.0, jnp.int32)
                iy1 = lax.convert_element_type(fy + 1.0, jnp.int32)
                iz1 = lax.convert_element_type(fz + 1.0, jnp.int32)
                ax0 = ix0 * (sp1 * sp1) + OFFS[l]
                ax1 = ix1 * (sp1 * sp1) + OFFS[l]
                by0 = iy0 * sp1
                by1 = iy1 * sp1
                for c in range(8):
                    ind = ((ax1 if (c & 4) else ax0)
                           + (by1 if (c & 2) else by0)
                           + (iz1 if (c & 1) else iz0))
                    idxbuf[l, pl.ds(c * 16, 16)] = ind
                for comp, src in enumerate((dx_hbm, dy_hbm, dz_hbm)):
                    cp = pltpu.make_async_copy(
                        src.at[idxbuf.at[l]], gbuf.at[l, comp], sem)
                    cp.start()
                    copies.append(cp)
                tx = fx - lax.convert_element_type(ix0, jnp.float32)
                ty = fy - lax.convert_element_type(iy0, jnp.float32)
                tz = fz - lax.convert_element_type(iz0, jnp.float32)
                lvl_cache.append((tx, ty, tz))

            for l in range(NLEV):
                for cpy in copies[3 * l:3 * l + 3]:
                    cpy.wait()
                tx, ty, tz = lvl_cache[l]
                wx = (jnp.clip(1.0 - tx, 0.0, 1.0), jnp.clip(tx, 0.0, 1.0))
                wy = (jnp.clip(1.0 - ty, 0.0, 1.0), jnp.clip(ty, 0.0, 1.0))
                wz = (jnp.clip(1.0 - tz, 0.0, 1.0), jnp.clip(tz, 0.0, 1.0))
                asx = jnp.zeros((16,), jnp.float32)
                asy = jnp.zeros((16,), jnp.float32)
                asz = jnp.zeros((16,), jnp.float32)
                acx = jnp.zeros((16,), jnp.float32)
                acy = jnp.zeros((16,), jnp.float32)
                acz = jnp.zeros((16,), jnp.float32)
                for c in range(8):
                    w = (wx[(c >> 2) & 1] * wy[(c >> 1) & 1]) * wz[c & 1]
                    vx = gbuf[l, 0, pl.ds(c * 16, 16)]
                    vy = gbuf[l, 1, pl.ds(c * 16, 16)]
                    vz = gbuf[l, 2, pl.ds(c * 16, 16)]
                    sx, cx = _sincos(vx * FREQ[l])
                    sy, cy = _sincos(vy * FREQ[l])
                    sz, cz = _sincos(vz * FREQ[l])
                    asx = asx + w * sx
                    asy = asy + w * sy
                    asz = asz + w * sz
                    acx = acx + w * cx
                    acy = acy + w * cy
                    acz = acz + w * cz
                col = 3 + 6 * l
                plsc.store_scatter(obuf, [iota, _splat_i32(col + 0)], asx)
                plsc.store_scatter(obuf, [iota, _splat_i32(col + 1)], asy)
                plsc.store_scatter(obuf, [iota, _splat_i32(col + 2)], asz)
                plsc.store_scatter(obuf, [iota, _splat_i32(col + 3)], acx)
                plsc.store_scatter(obuf, [iota, _splat_i32(col + 4)], acy)
                plsc.store_scatter(obuf, [iota, _splat_i32(col + 5)], acz)

            plsc.store_scatter(obuf, [iota, _splat_i32(0)], x)
            plsc.store_scatter(obuf, [iota, _splat_i32(1)], y)
            plsc.store_scatter(obuf, [iota, _splat_i32(2)], z)
            pltpu.sync_copy(obuf, out_hbm.at[pl.ds(wbase + base, MBS)])
            return _

        lax.fori_loop(0, NMB, mb_body, None)

    return sc_call


_sc_call_cache = []


def kernel(xyz, data):
    if not _sc_call_cache:
        _sc_call_cache.append(_make_sc_call())
    xyz_t = jnp.pad(xyz, ((0, NPAD - NPTS), (0, 0))).T
    one = xyz[0, 0] * 0.0 + 1.0
    dx = data[:, 0] * one
    dy = data[:, 1] * one
    dz = data[:, 2] * one
    out = _sc_call_cache[0](xyz_t, dx, dy, dz)
    return out[:NPTS]


# speedup vs baseline: 11.6825x; 1.3390x over previous
import functools

import numpy as np
import jax
import jax.numpy as jnp
from jax import lax
from jax.experimental import pallas as pl
from jax.experimental.pallas import tpu as pltpu
from jax.experimental.pallas import tpu_sc as plsc

NLEV = 8
SCALES = [16, 21, 28, 39, 52, 70, 95, 128]
_offs = [0]
for _s in SCALES:
    _offs.append(_offs[-1] + (_s + 1) ** 3)
OFFS = _offs
TOTAL = OFFS[-1]
FREQ = [float(2.0 ** l) for l in range(NLEV)]

NPTS = 100000
NW = 32
MBS = 16
PPW = 3200
NMB = PPW // MBS
NPAD = NW * PPW
OUTW = 3 + 6 * NLEV

BOUND_LO = np.float32(-1.0)
BOUND_HI = np.float32(np.float32(1.0) - np.float32(1e-6))

S0, S1, S2 = 0.99977134, -0.16582688, 0.00757417
C0, C1, C2, C3 = 0.99999528, -0.4999309, 0.04151171, -0.0012787
MAGIC = np.float32(1.5 * 2 ** 23)
INV_PI = float(1.0 / np.pi)
PI_HI = float(np.float32(np.pi))
PI_LO = float(np.float32(np.pi - np.float64(np.float32(np.pi))))


def _sincos(a):
    kf = a * INV_PI + MAGIC
    kbits = lax.bitcast_convert_type(kf, jnp.int32)
    kf = kf - MAGIC
    r = a - kf * PI_HI
    r = r - kf * PI_LO
    r2 = r * r
    ps = r * (S0 + r2 * (S1 + r2 * S2))
    pc = C0 + r2 * (C1 + r2 * (C2 + r2 * C3))
    sgn = lax.shift_left(lax.bitwise_and(kbits, 1), 31)
    s = lax.bitcast_convert_type(
        lax.bitwise_xor(lax.bitcast_convert_type(ps, jnp.int32), sgn), jnp.float32)
    c = lax.bitcast_convert_type(
        lax.bitwise_xor(lax.bitcast_convert_type(pc, jnp.int32), sgn), jnp.float32)
    return s, c


def _splat_i32(v):
    return jnp.full((16,), v, dtype=jnp.int32)


def _make_sc_call():
    mesh = plsc.VectorSubcoreMesh(core_axis_name="c", subcore_axis_name="s")

    @functools.partial(
        pl.kernel,
        mesh=mesh,
        out_type=jax.ShapeDtypeStruct((NPAD, OUTW), jnp.float32),
        compiler_params=pltpu.CompilerParams(
            needs_layout_passes=False, use_tc_tiling_on_sc=False),
        scratch_types=[
            pltpu.VMEM((3, PPW), jnp.float32),
            pltpu.VMEM((NLEV, 128), jnp.int32),
            pltpu.VMEM((NLEV, 128), jnp.int32),
            pltpu.VMEM((NLEV, 3, 128), jnp.float32),
            pltpu.VMEM((NLEV, 3, 128), jnp.float32),
            pltpu.VMEM((MBS, OUTW), jnp.float32),
            pltpu.SemaphoreType.DMA,
            pltpu.SemaphoreType.DMA,
        ],
    )
    def sc_call(xyz_hbm, dx_hbm, dy_hbm, dz_hbm, out_hbm,
                xbuf, idxA, idxB, gbufA, gbufB, obuf, semA, semB):
        wid = lax.axis_index("s") * 2 + lax.axis_index("c")
        wbase = wid * PPW
        for i in range(3):
            pltpu.sync_copy(xyz_hbm.at[i, pl.ds(wbase, PPW)], xbuf.at[i])

        iota = lax.broadcasted_iota(jnp.int32, (16,), 0)
        srcs = (dx_hbm, dy_hbm, dz_hbm)

        def load_xyz(m):
            base = m * MBS
            x = xbuf[0, pl.ds(base, MBS)]
            y = xbuf[1, pl.ds(base, MBS)]
            z = xbuf[2, pl.ds(base, MBS)]
            return x, y, z

        def normalize(x, y, z):
            xn = (jnp.minimum(jnp.maximum(x, BOUND_LO), BOUND_HI) + 1.0) * 0.5
            yn = (jnp.minimum(jnp.maximum(y, BOUND_LO), BOUND_HI) + 1.0) * 0.5
            zn = (jnp.minimum(jnp.maximum(z, BOUND_LO), BOUND_HI) + 1.0) * 0.5
            return xn, yn, zn

        def phase_a(m, idxbuf, sem, gbuf):
            x, y, z = load_xyz(m)
            xn, yn, zn = normalize(x, y, z)
            for l in range(NLEV):
                scale = float(SCALES[l])
                sp1 = SCALES[l] + 1
                fx = xn * scale
                fy = yn * scale
                fz = zn * scale
                ix0 = lax.convert_element_type(fx, jnp.int32)
                iy0 = lax.convert_element_type(fy, jnp.int32)
                iz0 = lax.convert_element_type(fz, jnp.int32)
                ix1 = lax.convert_element_type(fx + 1.0, jnp.int32)
                iy1 = lax.convert_element_type(fy + 1.0, jnp.int32)
                iz1 = lax.convert_element_type(fz + 1.0, jnp.int32)
                ax0 = ix0 * (sp1 * sp1) + OFFS[l]
                ax1 = ix1 * (sp1 * sp1) + OFFS[l]
                by0 = iy0 * sp1
                by1 = iy1 * sp1
                for c in range(8):
                    ind = ((ax1 if (c & 4) else ax0)
                           + (by1 if (c & 2) else by0)
                           + (iz1 if (c & 1) else iz0))
                    idxbuf[l, pl.ds(c * 16, 16)] = ind
                for comp in range(3):
                    pltpu.make_async_copy(
                        srcs[comp].at[idxbuf.at[l]], gbuf.at[l, comp], sem
                    ).start()

        def phase_b(m, idxbuf, sem, gbuf):
            x, y, z = load_xyz(m)
            xn, yn, zn = normalize(x, y, z)
            for l in range(NLEV):
                for comp in range(3):
                    pltpu.make_async_copy(
                        srcs[comp].at[idxbuf.at[l]], gbuf.at[l, comp], sem
                    ).wait()
                scale = float(SCALES[l])
                fx = xn * scale
                fy = yn * scale
                fz = zn * scale
                tx = fx - lax.convert_element_type(
                    lax.convert_element_type(fx, jnp.int32), jnp.float32)
                ty = fy - lax.convert_element_type(
                    lax.convert_element_type(fy, jnp.int32), jnp.float32)
                tz = fz - lax.convert_element_type(
                    lax.convert_element_type(fz, jnp.int32), jnp.float32)
                wx = (jnp.clip(1.0 - tx, 0.0, 1.0), jnp.clip(tx, 0.0, 1.0))
                wy = (jnp.clip(1.0 - ty, 0.0, 1.0), jnp.clip(ty, 0.0, 1.0))
                wz = (jnp.clip(1.0 - tz, 0.0, 1.0), jnp.clip(tz, 0.0, 1.0))
                asx = jnp.zeros((16,), jnp.float32)
                asy = jnp.zeros((16,), jnp.float32)
                asz = jnp.zeros((16,), jnp.float32)
                acx = jnp.zeros((16,), jnp.float32)
                acy = jnp.zeros((16,), jnp.float32)
                acz = jnp.zeros((16,), jnp.float32)
                for c in range(8):
                    w = (wx[(c >> 2) & 1] * wy[(c >> 1) & 1]) * wz[c & 1]
                    vx = gbuf[l, 0, pl.ds(c * 16, 16)]
                    vy = gbuf[l, 1, pl.ds(c * 16, 16)]
                    vz = gbuf[l, 2, pl.ds(c * 16, 16)]
                    sx, cx = _sincos(vx * FREQ[l])
                    sy, cy = _sincos(vy * FREQ[l])
                    sz, cz = _sincos(vz * FREQ[l])
                    asx = asx + w * sx
                    asy = asy + w * sy
                    asz = asz + w * sz
                    acx = acx + w * cx
                    acy = acy + w * cy
                    acz = acz + w * cz
                col = 3 + 6 * l
                plsc.store_scatter(obuf, [iota, _splat_i32(col + 0)], asx)
                plsc.store_scatter(obuf, [iota, _splat_i32(col + 1)], asy)
                plsc.store_scatter(obuf, [iota, _splat_i32(col + 2)], asz)
                plsc.store_scatter(obuf, [iota, _splat_i32(col + 3)], acx)
                plsc.store_scatter(obuf, [iota, _splat_i32(col + 4)], acy)
                plsc.store_scatter(obuf, [iota, _splat_i32(col + 5)], acz)
            plsc.store_scatter(obuf, [iota, _splat_i32(0)], x)
            plsc.store_scatter(obuf, [iota, _splat_i32(1)], y)
            plsc.store_scatter(obuf, [iota, _splat_i32(2)], z)
            pltpu.sync_copy(obuf, out_hbm.at[pl.ds(wbase + m * MBS, MBS)])

        phase_a(0, idxA, semA, gbufA)

        def pair_body(i, _):
            phase_a(2 * i + 1, idxB, semB, gbufB)
            phase_b(2 * i, idxA, semA, gbufA)

            @pl.when(i < NMB // 2 - 1)
            def _():
                phase_a(2 * i + 2, idxA, semA, gbufA)

            phase_b(2 * i + 1, idxB, semB, gbufB)
            return _

        lax.fori_loop(0, NMB // 2, pair_body, None)

    return sc_call


_sc_call_cache = []


def kernel(xyz, data):
    if not _sc_call_cache:
        _sc_call_cache.append(_make_sc_call())
    xyz_t = jnp.pad(xyz, ((0, NPAD - NPTS), (0, 0))).T
    one = xyz[0, 0] * 0.0 + 1.0
    dx = data[:, 0] * one
    dy = data[:, 1] * one
    dz = data[:, 2] * one
    out = _sc_call_cache[0](xyz_t, dx, dy, dz)
    return out[:NPTS]
